# baseline (speedup 1.0000x reference)
import functools

import jax
import jax.numpy as jnp
from jax import lax
from jax.experimental import pallas as pl
from jax.experimental.pallas import tpu as pltpu
from jax.experimental.pallas import tpu_sc as plsc

N = 10000
D = 128
E = 320000
ETOT = E + N
EPS = 1e-5

NC, NS, L = 2, 16, 16
NW = NC * NS
K = 128
NBUF = 2
BLOCKS = 84
C = K * BLOCKS
EPAD = C * NW
NPAD = 10240
RPT = NPAD // NS
ZR = 16

@functools.cache
def _mesh():
    return plsc.VectorSubcoreMesh(
        core_axis_name="c", subcore_axis_name="s",
        num_cores=NC, num_subcores=NS)


def _degree_body(dstr, out, didx, ones, buf, acc):
    c = lax.axis_index("c")
    s = lax.axis_index("s")
    wid = c * NS + s
    row0 = s * RPT

    def zloop(i, carry):
        buf[pl.ds(i * L, L)] = jnp.zeros((L,), jnp.float32)
        return carry
    lax.fori_loop(0, RPT // L, zloop, 0)

    def oloop(i, carry):
        ones[pl.ds(i * L, L)] = jnp.ones((L,), jnp.float32)
        return carry
    lax.fori_loop(0, K // L, oloop, 0)

    pltpu.sync_copy(buf, acc.at[pl.ds(row0, RPT)])
    plsc.subcore_barrier()

    def body(j, carry):
        eoff = wid * C + j * K
        pltpu.sync_copy(dstr.at[pl.ds(eoff, K)], didx)
        pltpu.sync_copy(ones, acc.at[didx], add=True)
        return carry
    lax.fori_loop(0, BLOCKS, body, 0)
    plsc.subcore_barrier()

    pltpu.sync_copy(acc.at[pl.ds(row0, RPT)], buf)
    pltpu.sync_copy(buf, out.at[c, pl.ds(row0, RPT)])


@functools.cache
def _sc_degree():
    return pl.kernel(
        _degree_body,
        out_type=jax.ShapeDtypeStruct((NC, NPAD), jnp.float32),
        mesh=_mesh(),
        scratch_types=[
            pltpu.VMEM((K,), jnp.int32),
            pltpu.VMEM((K,), jnp.float32),
            pltpu.VMEM((RPT,), jnp.float32),
            pltpu.VMEM_SHARED((NPAD,), jnp.float32),
        ],
    )


def _scatter_body(hp, srcr, dstr, out, sidx, didx, rows, zbuf, acc, *sems):
    gsem = sems[:NBUF]
    ssem = sems[NBUF:]
    c = lax.axis_index("c")
    s = lax.axis_index("s")
    wid = c * NS + s
    row0 = s * RPT

    def zloop(i, carry):
        zbuf[i // 8, pl.ds((i % 8) * L, L)] = jnp.zeros((L,), jnp.float32)
        return carry
    lax.fori_loop(0, ZR * (D // L), zloop, 0)

    def zcopy(j, carry):
        pltpu.sync_copy(zbuf, acc.at[pl.ds(row0 + j * ZR, ZR)])
        return carry
    lax.fori_loop(0, RPT // ZR, zcopy, 0)
    plsc.subcore_barrier()

    def body(j, carry):
        eoff = wid * C + j * K
        pltpu.sync_copy(srcr.at[pl.ds(eoff, K)], sidx)
        pltpu.sync_copy(dstr.at[pl.ds(eoff, K)], didx)
        pltpu.async_copy(hp.at[sidx], rows.at[0], gsem[0]).wait()
        pltpu.sync_copy(rows.at[0], acc.at[didx], add=True)
        return carry
    lax.fori_loop(0, BLOCKS, body, 0)
    plsc.subcore_barrier()

    OB = 80
    NOB = RPT // OB
    for b in range(NBUF):
        r0 = row0 + b * OB
        pltpu.async_copy(acc.at[pl.ds(r0, OB)], rows.at[b, pl.ds(0, OB)],
                         gsem[b])
    for jj in range(NOB):
        b = jj % NBUF
        r = row0 + jj * OB
        pltpu.make_async_copy(acc.at[pl.ds(r, OB)], rows.at[b, pl.ds(0, OB)],
                              gsem[b]).wait()
        pltpu.async_copy(rows.at[b, pl.ds(0, OB)], out.at[c, pl.ds(r, OB)],
                         ssem[b])
        nxt = jj + NBUF
        pltpu.make_async_copy(rows.at[b, pl.ds(0, OB)],
                              out.at[c, pl.ds(r, OB)], ssem[b]).wait()
        if nxt < NOB:
            rn = row0 + nxt * OB
            pltpu.async_copy(acc.at[pl.ds(rn, OB)], rows.at[b, pl.ds(0, OB)],
                             gsem[b])


@functools.cache
def _sc_scatter():
    return pl.kernel(
        _scatter_body,
        out_type=jax.ShapeDtypeStruct((NC, NPAD, D), jnp.float32),
        mesh=_mesh(),
        scratch_types=[
            pltpu.VMEM((K,), jnp.int32),
            pltpu.VMEM((K,), jnp.int32),
            pltpu.VMEM((NBUF, K, D), jnp.float32),
            pltpu.VMEM((ZR, D), jnp.float32),
            pltpu.VMEM_SHARED((NPAD, D), jnp.float32),
        ] + [pltpu.SemaphoreType.DMA] * (2 * NBUF),
    )


def _pre_body(deg_ref, x_ref, w_ref, hp_ref, dinv_ref):
    deg = deg_ref[0] + deg_ref[1]
    rows = lax.broadcasted_iota(jnp.int32, (NPAD, 1), 0)
    dinv = jnp.where(rows < N, lax.rsqrt(jnp.maximum(deg, 1.0)), 0.0)
    hp = jnp.dot(x_ref[...], w_ref[...], preferred_element_type=jnp.float32)
    hp_ref[...] = hp * dinv
    dinv_ref[...] = dinv


def _mid_body(s_ref, dinv_ref, b_ref, g_ref, bt_ref, w_ref, hp_ref):
    sacc = s_ref[0] + s_ref[1]
    dinv = dinv_ref[...]
    cfull = sacc * dinv + b_ref[...]
    cn = cfull[:N]
    mean = jnp.mean(cn, axis=0, keepdims=True)
    xc = cn - mean
    var = jnp.mean(xc * xc, axis=0, keepdims=True)
    scale = g_ref[...] * lax.rsqrt(var + EPS)
    zfull = (cfull - mean) * scale + bt_ref[...]
    hp = jnp.dot(zfull, w_ref[...], preferred_element_type=jnp.float32)
    hp_ref[...] = hp * dinv


def _post_body(s_ref, dinv_ref, b_ref, out_ref):
    sacc = s_ref[0] + s_ref[1]
    out_ref[...] = (sacc * dinv_ref[...] + b_ref[...])[:N]


def _tc_pre(deg2, x_pad, w1):
    return pl.pallas_call(
        _pre_body,
        out_shape=[jax.ShapeDtypeStruct((NPAD, D), jnp.float32),
                   jax.ShapeDtypeStruct((NPAD, 1), jnp.float32)],
    )(deg2, x_pad, w1)


def _tc_mid(s2, dinv, b, g, bt, w):
    return pl.pallas_call(
        _mid_body,
        out_shape=jax.ShapeDtypeStruct((NPAD, D), jnp.float32),
    )(s2, dinv, b, g, bt, w)


def _tc_post(s2, dinv, b):
    return pl.pallas_call(
        _post_body,
        out_shape=jax.ShapeDtypeStruct((N, D), jnp.float32),
    )(s2, dinv, b)


def kernel(x, edge_index, W1, b1, g2, bt2, Wm, bm, g3, bt3, W2, b2):
    loops = jnp.arange(N, dtype=jnp.int32)
    pad_src = jnp.full((EPAD - ETOT,), N, dtype=jnp.int32)
    pad_dst = N + jnp.arange(EPAD - ETOT, dtype=jnp.int32) % (NPAD - N)
    src = jnp.concatenate([edge_index[0], loops, pad_src])
    dst = jnp.concatenate([edge_index[1], loops, pad_dst])
    x_pad = jnp.pad(x, ((0, NPAD - N), (0, 0)))

    deg2 = _sc_degree()(dst)
    hp1, dinv = _tc_pre(deg2[:, :, None], x_pad, W1)
    scat = _sc_scatter()
    s1 = scat(hp1, src, dst)
    hp2 = _tc_mid(s1, dinv, b1[None, :], g2[None, :], bt2[None, :], Wm)
    s2 = scat(hp2, src, dst)
    hp3 = _tc_mid(s2, dinv, bm[None, :], g3[None, :], bt3[None, :], W2)
    s3 = scat(hp3, src, dst)
    return _tc_post(s3, dinv, b2[None, :])


# speedup vs baseline: 1.1041x; 1.1041x over previous
import functools

import jax
import jax.numpy as jnp
from jax import lax
from jax.experimental import pallas as pl
from jax.experimental.pallas import tpu as pltpu
from jax.experimental.pallas import tpu_sc as plsc

N = 10000
D = 128
E = 320000
ETOT = E + N
EPS = 1e-5

NC, NS, L = 2, 16, 16
NW = NC * NS
K = 128
NBUF = 2
BLOCKS = 84
C = K * BLOCKS
EPAD = C * NW
NPAD = 10240
RPT = NPAD // NS
ZR = 64

@functools.cache
def _mesh():
    return plsc.VectorSubcoreMesh(
        core_axis_name="c", subcore_axis_name="s",
        num_cores=NC, num_subcores=NS)


def _degree_body(dstr, out, didx, ones, buf, acc):
    c = lax.axis_index("c")
    s = lax.axis_index("s")
    wid = c * NS + s
    row0 = s * RPT

    def zloop(i, carry):
        buf[pl.ds(i * L, L)] = jnp.zeros((L,), jnp.float32)
        return carry
    lax.fori_loop(0, RPT // L, zloop, 0)

    def oloop(i, carry):
        ones[pl.ds(i * L, L)] = jnp.ones((L,), jnp.float32)
        return carry
    lax.fori_loop(0, K // L, oloop, 0)

    pltpu.sync_copy(buf, acc.at[pl.ds(row0, RPT)])
    plsc.subcore_barrier()

    def body(j, carry):
        eoff = wid * C + j * K
        pltpu.sync_copy(dstr.at[pl.ds(eoff, K)], didx)
        pltpu.sync_copy(ones, acc.at[didx], add=True)
        return carry
    lax.fori_loop(0, BLOCKS, body, 0)
    plsc.subcore_barrier()

    pltpu.sync_copy(acc.at[pl.ds(row0, RPT)], buf)
    pltpu.sync_copy(buf, out.at[c, pl.ds(row0, RPT)])


@functools.cache
def _sc_degree():
    return pl.kernel(
        _degree_body,
        out_type=jax.ShapeDtypeStruct((NC, NPAD), jnp.float32),
        mesh=_mesh(),
        scratch_types=[
            pltpu.VMEM((K,), jnp.int32),
            pltpu.VMEM((K,), jnp.float32),
            pltpu.VMEM((RPT,), jnp.float32),
            pltpu.VMEM_SHARED((NPAD,), jnp.float32),
        ],
    )


def _scatter_body(hp, srcr, dstr, out,
                  sidx0, sidx1, didx0, didx1, rows0, rows1, zbuf, acc,
                  gsem0, gsem1):
    sidx = (sidx0, sidx1)
    didx = (didx0, didx1)
    rows = (rows0, rows1)
    gsem = (gsem0, gsem1)
    c = lax.axis_index("c")
    s = lax.axis_index("s")
    wid = c * NS + s
    row0 = s * RPT

    def zloop(i, carry):
        zbuf[i // 8, pl.ds((i % 8) * L, L)] = jnp.zeros((L,), jnp.float32)
        return carry
    lax.fori_loop(0, ZR * (D // L), zloop, 0)

    def zcopy(j, carry):
        pltpu.sync_copy(zbuf, acc.at[pl.ds(row0 + j * ZR, ZR)])
        return carry
    lax.fori_loop(0, RPT // ZR, zcopy, 0)
    plsc.subcore_barrier()

    def load_and_gather(j, b):
        eoff = wid * C + j * K
        pltpu.sync_copy(srcr.at[pl.ds(eoff, K)], sidx[b])
        pltpu.sync_copy(dstr.at[pl.ds(eoff, K)], didx[b])
        pltpu.async_copy(hp.at[sidx[b]], rows[b], gsem[b])

    def wait_gather(b):
        pltpu.make_async_copy(hp.at[sidx[b]], rows[b], gsem[b]).wait()

    for b in range(NBUF):
        load_and_gather(b, b)

    def body(sb, carry):
        j0 = sb * NBUF
        for b in range(NBUF):
            wait_gather(b)
            pltpu.sync_copy(rows[b], acc.at[didx[b]], add=True)
            load_and_gather(j0 + NBUF + b, b)
        return carry
    lax.fori_loop(0, BLOCKS // NBUF - 1, body, 0)

    for b in range(NBUF):
        wait_gather(b)
        pltpu.sync_copy(rows[b], acc.at[didx[b]], add=True)
    plsc.subcore_barrier()

    def ocopy(j, carry):
        r = row0 + j * ZR
        pltpu.sync_copy(acc.at[pl.ds(r, ZR)], zbuf)
        pltpu.sync_copy(zbuf, out.at[c, pl.ds(r, ZR)])
        return carry
    lax.fori_loop(0, RPT // ZR, ocopy, 0)


@functools.cache
def _sc_scatter():
    return pl.kernel(
        _scatter_body,
        out_type=jax.ShapeDtypeStruct((NC, NPAD, D), jnp.float32),
        mesh=_mesh(),
        scratch_types=[
            pltpu.VMEM((K,), jnp.int32),
            pltpu.VMEM((K,), jnp.int32),
            pltpu.VMEM((K,), jnp.int32),
            pltpu.VMEM((K,), jnp.int32),
            pltpu.VMEM((K, D), jnp.float32),
            pltpu.VMEM((K, D), jnp.float32),
            pltpu.VMEM((ZR, D), jnp.float32),
            pltpu.VMEM_SHARED((NPAD, D), jnp.float32),
            pltpu.SemaphoreType.DMA,
            pltpu.SemaphoreType.DMA,
        ],
    )


def _pre_body(deg_ref, x_ref, w_ref, hp_ref, dinv_ref):
    deg = deg_ref[0] + deg_ref[1]
    rows = lax.broadcasted_iota(jnp.int32, (NPAD, 1), 0)
    dinv = jnp.where(rows < N, lax.rsqrt(jnp.maximum(deg, 1.0)), 0.0)
    hp = jnp.dot(x_ref[...], w_ref[...], preferred_element_type=jnp.float32)
    hp_ref[...] = hp * dinv
    dinv_ref[...] = dinv


def _mid_body(s_ref, dinv_ref, b_ref, g_ref, bt_ref, w_ref, hp_ref):
    sacc = s_ref[0] + s_ref[1]
    dinv = dinv_ref[...]
    cfull = sacc * dinv + b_ref[...]
    cn = cfull[:N]
    mean = jnp.mean(cn, axis=0, keepdims=True)
    xc = cn - mean
    var = jnp.mean(xc * xc, axis=0, keepdims=True)
    scale = g_ref[...] * lax.rsqrt(var + EPS)
    zfull = (cfull - mean) * scale + bt_ref[...]
    hp = jnp.dot(zfull, w_ref[...], preferred_element_type=jnp.float32)
    hp_ref[...] = hp * dinv


def _post_body(s_ref, dinv_ref, b_ref, out_ref):
    sacc = s_ref[0] + s_ref[1]
    out_ref[...] = (sacc * dinv_ref[...] + b_ref[...])[:N]


def _tc_pre(deg2, x_pad, w1):
    return pl.pallas_call(
        _pre_body,
        out_shape=[jax.ShapeDtypeStruct((NPAD, D), jnp.float32),
                   jax.ShapeDtypeStruct((NPAD, 1), jnp.float32)],
    )(deg2, x_pad, w1)


def _tc_mid(s2, dinv, b, g, bt, w):
    return pl.pallas_call(
        _mid_body,
        out_shape=jax.ShapeDtypeStruct((NPAD, D), jnp.float32),
    )(s2, dinv, b, g, bt, w)


def _tc_post(s2, dinv, b):
    return pl.pallas_call(
        _post_body,
        out_shape=jax.ShapeDtypeStruct((N, D), jnp.float32),
    )(s2, dinv, b)


def kernel(x, edge_index, W1, b1, g2, bt2, Wm, bm, g3, bt3, W2, b2):
    loops = jnp.arange(N, dtype=jnp.int32)
    pad_src = jnp.full((EPAD - ETOT,), N, dtype=jnp.int32)
    pad_dst = N + jnp.arange(EPAD - ETOT, dtype=jnp.int32) % (NPAD - N)
    src = jnp.concatenate([edge_index[0], loops, pad_src])
    dst = jnp.concatenate([edge_index[1], loops, pad_dst])
    x_pad = jnp.pad(x, ((0, NPAD - N), (0, 0)))

    deg2 = _sc_degree()(dst)
    hp1, dinv = _tc_pre(deg2[:, :, None], x_pad, W1)
    scat = _sc_scatter()
    s1 = scat(hp1, src, dst)
    hp2 = _tc_mid(s1, dinv, b1[None, :], g2[None, :], bt2[None, :], Wm)
    s2 = scat(hp2, src, dst)
    hp3 = _tc_mid(s2, dinv, bm[None, :], g3[None, :], bt3[None, :], W2)
    s3 = scat(hp3, src, dst)
    return _tc_post(s3, dinv, b2[None, :])


# speedup vs baseline: 5.5927x; 5.0653x over previous
import functools

import jax
import jax.numpy as jnp
from jax import lax
from jax.experimental import pallas as pl
from jax.experimental.pallas import tpu as pltpu
from jax.experimental.pallas import tpu_sc as plsc

N = 10000
D = 128
E = 320000
ETOT = E + N
EPS = 1e-5

NC, NS, L = 2, 16, 16
NW = NC * NS
K = 128
NBUF = 2
BLOCKS = 82
C = K * BLOCKS
EPAD = C * NW
NPAD = 10240
RPT = NPAD // NS
ZR = 64

@functools.cache
def _mesh():
    return plsc.VectorSubcoreMesh(
        core_axis_name="c", subcore_axis_name="s",
        num_cores=NC, num_subcores=NS)


def _degree_body(dstr, out, didx, ones, buf, acc):
    c = lax.axis_index("c")
    s = lax.axis_index("s")
    wid = c * NS + s
    row0 = s * RPT

    def zloop(i, carry):
        buf[pl.ds(i * L, L)] = jnp.zeros((L,), jnp.float32)
        return carry
    lax.fori_loop(0, RPT // L, zloop, 0)

    def oloop(i, carry):
        ones[pl.ds(i * L, L)] = jnp.ones((L,), jnp.float32)
        return carry
    lax.fori_loop(0, K // L, oloop, 0)

    pltpu.sync_copy(buf, acc.at[pl.ds(row0, RPT)])
    plsc.subcore_barrier()

    def body(j, carry):
        eoff = wid * C + j * K
        pltpu.sync_copy(dstr.at[pl.ds(eoff, K)], didx)
        pltpu.sync_copy(ones, acc.at[didx], add=True)
        return carry
    lax.fori_loop(0, BLOCKS, body, 0)
    plsc.subcore_barrier()

    pltpu.sync_copy(acc.at[pl.ds(row0, RPT)], buf)
    pltpu.sync_copy(buf, out.at[c, pl.ds(row0, RPT)])


@functools.cache
def _sc_degree():
    return pl.kernel(
        _degree_body,
        out_type=jax.ShapeDtypeStruct((NC, NPAD), jnp.float32),
        mesh=_mesh(),
        scratch_types=[
            pltpu.VMEM((K,), jnp.int32),
            pltpu.VMEM((K,), jnp.float32),
            pltpu.VMEM((RPT,), jnp.float32),
            pltpu.VMEM_SHARED((NPAD,), jnp.float32),
        ],
    )


def _scatter_body(hp, srcr, dstr, out,
                  sidx0, sidx1, didx0, didx1, rows0, rows1, zbuf, acc,
                  gsem0, gsem1):
    sidx = (sidx0, sidx1)
    didx = (didx0, didx1)
    rows = (rows0, rows1)
    gsem = (gsem0, gsem1)
    c = lax.axis_index("c")
    s = lax.axis_index("s")
    wid = c * NS + s
    row0 = s * RPT

    def zloop(i, carry):
        zbuf[i // 8, pl.ds((i % 8) * L, L)] = jnp.zeros((L,), jnp.float32)
        return carry
    lax.fori_loop(0, ZR * (D // L), zloop, 0)

    def zcopy(j, carry):
        pltpu.sync_copy(zbuf, acc.at[pl.ds(row0 + j * ZR, ZR)])
        return carry
    lax.fori_loop(0, RPT // ZR, zcopy, 0)
    plsc.subcore_barrier()

    def load_and_gather(j, b):
        eoff = wid * C + j * K
        pltpu.sync_copy(srcr.at[pl.ds(eoff, K)], sidx[b])
        pltpu.sync_copy(dstr.at[pl.ds(eoff, K)], didx[b])
        pltpu.async_copy(hp.at[sidx[b]], rows[b], gsem[b])

    def wait_gather(b):
        pltpu.make_async_copy(hp.at[sidx[b]], rows[b], gsem[b]).wait()

    for b in range(NBUF):
        load_and_gather(b, b)

    def body(sb, carry):
        j0 = sb * NBUF
        for b in range(NBUF):
            wait_gather(b)
            pltpu.sync_copy(rows[b], acc.at[didx[b]], add=True)
            load_and_gather(j0 + NBUF + b, b)
        return carry
    lax.fori_loop(0, BLOCKS // NBUF - 1, body, 0)

    for b in range(NBUF):
        wait_gather(b)
        pltpu.sync_copy(rows[b], acc.at[didx[b]], add=True)
    plsc.subcore_barrier()

    def ocopy(j, carry):
        r = row0 + j * ZR
        pltpu.sync_copy(acc.at[pl.ds(r, ZR)], zbuf)
        pltpu.sync_copy(zbuf, out.at[c, pl.ds(r, ZR)])
        return carry
    lax.fori_loop(0, RPT // ZR, ocopy, 0)


@functools.cache
def _sc_scatter():
    return pl.kernel(
        _scatter_body,
        out_type=jax.ShapeDtypeStruct((NC, NPAD, D), jnp.float32),
        mesh=_mesh(),
        scratch_types=[
            pltpu.VMEM((K,), jnp.int32),
            pltpu.VMEM((K,), jnp.int32),
            pltpu.VMEM((K,), jnp.int32),
            pltpu.VMEM((K,), jnp.int32),
            pltpu.VMEM((K, D), jnp.float32),
            pltpu.VMEM((K, D), jnp.float32),
            pltpu.VMEM((ZR, D), jnp.float32),
            pltpu.VMEM_SHARED((NPAD, D), jnp.float32),
            pltpu.SemaphoreType.DMA,
            pltpu.SemaphoreType.DMA,
        ],
    )


def _pre_body(deg_ref, x_ref, w_ref, hp_ref, dinv_ref):
    deg = deg_ref[0] + deg_ref[1]
    rows = lax.broadcasted_iota(jnp.int32, (NPAD, 1), 0)
    dinv = jnp.where(rows < N, lax.rsqrt(jnp.maximum(deg, 1.0)), 0.0)
    hp = jnp.dot(x_ref[...], w_ref[...], preferred_element_type=jnp.float32)
    hp_ref[...] = hp * dinv
    dinv_ref[...] = dinv


def _mid_body(s_ref, dinv_ref, b_ref, g_ref, bt_ref, w_ref, hp_ref):
    sacc = s_ref[0] + s_ref[1]
    dinv = dinv_ref[...]
    cfull = sacc * dinv + b_ref[...]
    cn = cfull[:N]
    mean = jnp.mean(cn, axis=0, keepdims=True)
    xc = cn - mean
    var = jnp.mean(xc * xc, axis=0, keepdims=True)
    scale = g_ref[...] * lax.rsqrt(var + EPS)
    zfull = (cfull - mean) * scale + bt_ref[...]
    hp = jnp.dot(zfull, w_ref[...], preferred_element_type=jnp.float32)
    hp_ref[...] = hp * dinv


def _post_body(s_ref, dinv_ref, b_ref, out_ref):
    sacc = s_ref[0] + s_ref[1]
    out_ref[...] = (sacc * dinv_ref[...] + b_ref[...])[:N]


def _tc_pre(deg2, x_pad, w1):
    return pl.pallas_call(
        _pre_body,
        out_shape=[jax.ShapeDtypeStruct((NPAD, D), jnp.float32),
                   jax.ShapeDtypeStruct((NPAD, 1), jnp.float32)],
    )(deg2, x_pad, w1)


def _tc_mid(s2, dinv, b, g, bt, w):
    return pl.pallas_call(
        _mid_body,
        out_shape=jax.ShapeDtypeStruct((NPAD, D), jnp.float32),
    )(s2, dinv, b, g, bt, w)


def _tc_post(s2, dinv, b):
    return pl.pallas_call(
        _post_body,
        out_shape=jax.ShapeDtypeStruct((N, D), jnp.float32),
    )(s2, dinv, b)


def kernel(x, edge_index, W1, b1, g2, bt2, Wm, bm, g3, bt3, W2, b2):
    loops = jnp.arange(N, dtype=jnp.int32)
    pad_src = N + jnp.arange(EPAD - ETOT, dtype=jnp.int32) % (NPAD - N)
    pad_dst = N + jnp.arange(EPAD - ETOT, dtype=jnp.int32) % (NPAD - N)
    src = jnp.concatenate([edge_index[0], loops, pad_src])
    dst = jnp.concatenate([edge_index[1], loops, pad_dst])
    x_pad = jnp.pad(x, ((0, NPAD - N), (0, 0)))

    deg2 = _sc_degree()(dst)
    hp1, dinv = _tc_pre(deg2[:, :, None], x_pad, W1)
    scat = _sc_scatter()
    s1 = scat(hp1, src, dst)
    hp2 = _tc_mid(s1, dinv, b1[None, :], g2[None, :], bt2[None, :], Wm)
    s2 = scat(hp2, src, dst)
    hp3 = _tc_mid(s2, dinv, bm[None, :], g3[None, :], bt3[None, :], W2)
    s3 = scat(hp3, src, dst)
    return _tc_post(s3, dinv, b2[None, :])
